# baseline (speedup 1.0000x reference)
import functools

import jax
import jax.numpy as jnp
from jax import lax
from jax.experimental import pallas as pl
from jax.experimental.pallas import tpu as pltpu
from jax.experimental.pallas import tpu_sc as plsc

D = 32
NBUF = 2


def kernel(x, table):
    R, C = x.shape
    info = plsc.get_sparse_core_info()
    NC, NS = info.num_cores, info.num_subcores
    NW = NC * NS
    IB = R // NW
    NT = D // 8
    xT = x.T

    mesh = plsc.VectorSubcoreMesh(core_axis_name="c", subcore_axis_name="s")

    @functools.partial(
        pl.kernel,
        mesh=mesh,
        out_type=jax.ShapeDtypeStruct((C * NT * NW * 8, IB), jnp.float32),
        scratch_types=[
            pltpu.VMEM((C, IB), jnp.int32),
            pltpu.VMEM((IB, D), jnp.float32),
            pltpu.VMEM((IB, D), jnp.float32),
            pltpu.VMEM((D, IB), jnp.float32),
            pltpu.VMEM((D, IB), jnp.float32),
        ]
        + [pltpu.SemaphoreType.DMA] * (2 * NBUF),
        compiler_params=pltpu.CompilerParams(
            use_tc_tiling_on_sc=False, needs_layout_passes=False
        ),
    )
    def emb(table_hbm, xT_hbm, out_hbm, idx_v, rows0, rows1, tr0, tr1, *sems):
        rows = (rows0, rows1)
        trs = (tr0, tr1)
        gsems = sems[:NBUF]
        wsems = sems[NBUF:]
        wid = lax.axis_index("s") * NC + lax.axis_index("c")
        pltpu.sync_copy(xT_hbm.at[:, pl.ds(wid * IB, IB)], idx_v)
        iota = lax.iota(jnp.int32, 16)
        row_vecs = [iota + ir0 for ir0 in range(0, IB, 16)]
        col_vecs = [jnp.full((16,), d, jnp.int32) for d in range(D)]

        def fire_g(j, b):
            pltpu.async_copy(table_hbm.at[idx_v.at[j]], rows[b], gsems[b])

        def drain_g(b):
            pltpu.make_async_copy(
                table_hbm.at[idx_v.at[0]], rows[b], gsems[b]
            ).wait()

        def transpose(b):
            src = rows[b]
            dst = trs[b]
            for ri, row_idx in enumerate(row_vecs):
                for d in range(D):
                    vals = plsc.load_gather(src, [row_idx, col_vecs[d]])
                    dst[d, pl.ds(ri * 16, 16)] = vals

        def fire_w(j, b):
            for dt in range(NT):
                pltpu.async_copy(
                    trs[b].at[pl.ds(dt * 8, 8)],
                    out_hbm.at[pl.ds(((j * NT + dt) * NW + wid) * 8, 8)],
                    wsems[b],
                )

        def drain_w(b):
            for dt in range(NT):
                pltpu.make_async_copy(
                    trs[b].at[pl.ds(dt * 8, 8)],
                    out_hbm.at[pl.ds(0, 8)],
                    wsems[b],
                ).wait()

        fire_g(0, 0)
        fire_g(1, 1)
        for b in range(NBUF):
            drain_g(b)
            transpose(b)
            fire_g(b + 2, b)
            fire_w(b, b)

        def body(p, carry):
            j0 = p * 2
            for b in range(NBUF):
                j = j0 + b
                drain_g(b)
                drain_w(b)
                transpose(b)
                fire_g(j + 2, b)
                fire_w(j, b)
            return carry

        lax.fori_loop(1, C // 2 - 1, body, 0)

        for b in range(NBUF):
            j = C - 2 + b
            drain_g(b)
            drain_w(b)
            transpose(b)
            fire_w(j, b)
        for b in range(NBUF):
            drain_w(b)

    out2 = emb(table, xT)
    f5 = out2.reshape(C, NT, NW, 8, IB)
    return f5.transpose(2, 4, 0, 1, 3).reshape(R, C, D)


# speedup vs baseline: 1.2217x; 1.2217x over previous
import functools

import jax
import jax.numpy as jnp
from jax import lax
from jax.experimental import pallas as pl
from jax.experimental.pallas import tpu as pltpu
from jax.experimental.pallas import tpu_sc as plsc

D = 32
NBUF = 4


def kernel(x, table):
    R, C = x.shape
    info = plsc.get_sparse_core_info()
    NC, NS = info.num_cores, info.num_subcores
    NW = NC * NS
    r_per_w = R // NW
    n_rounds = r_per_w // NBUF

    mesh = plsc.VectorSubcoreMesh(core_axis_name="c", subcore_axis_name="s")

    @functools.partial(
        pl.kernel,
        mesh=mesh,
        out_type=jax.ShapeDtypeStruct((R, C, D), jnp.float32),
        scratch_types=[
            pltpu.VMEM((r_per_w, C), jnp.int32),
            pltpu.VMEM((NBUF, C, D), jnp.float32),
        ]
        + [pltpu.SemaphoreType.DMA] * (2 * NBUF),
        compiler_params=pltpu.CompilerParams(use_tc_tiling_on_sc=False),
    )
    def emb(table_hbm, x_hbm, out_hbm, idx_v, rows_v, *sems):
        gsems = sems[:NBUF]
        wsems = sems[NBUF:]
        wid = lax.axis_index("s") * NC + lax.axis_index("c")
        base = wid * r_per_w
        pltpu.sync_copy(x_hbm.at[pl.ds(base, r_per_w)], idx_v)

        def fire_g(g, b):
            pltpu.async_copy(table_hbm.at[idx_v.at[g]], rows_v.at[b], gsems[b])

        def drain_g(b):
            pltpu.make_async_copy(
                table_hbm.at[idx_v.at[0]], rows_v.at[b], gsems[b]
            ).wait()

        def fire_w(g, b):
            pltpu.async_copy(rows_v.at[b], out_hbm.at[base + g], wsems[b])

        def drain_w(b):
            pltpu.make_async_copy(
                rows_v.at[b], out_hbm.at[base], wsems[b]
            ).wait()

        for b in range(NBUF):
            fire_g(b, b)

        def body(it, carry):
            g0 = it * NBUF
            for b in range(NBUF):
                g = g0 + b
                drain_g(b)
                fire_w(g, b)
                drain_w(b)
                fire_g(g + NBUF, b)
            return carry

        lax.fori_loop(0, n_rounds - 1, body, 0)

        g0 = (n_rounds - 1) * NBUF
        for b in range(NBUF):
            drain_g(b)
            fire_w(g0 + b, b)
        for b in range(NBUF):
            drain_w(b)

    return emb(table, x)
